# baseline (speedup 1.0000x reference)
import functools

import jax
import jax.numpy as jnp
from jax import lax
from jax.experimental import pallas as pl
from jax.experimental.pallas import tpu as pltpu
from jax.experimental.pallas import tpu_sc as plsc

N_BASIS = 4
SEGS = 512
IN_F = 32
OUT_F = 32
BATCH = 4096
N_COEF = (N_BASIS - 1) * SEGS + 1
TBL_ROWS = IN_F * N_COEF // 4

NW = 32
B_PER_W = BATCH // NW
LOOKUPS = B_PER_W * IN_F
NBUF = 4
LANES = 16

_mesh = plsc.VectorSubcoreMesh(core_axis_name="c", subcore_axis_name="s")


@functools.partial(
    pl.kernel,
    mesh=_mesh,
    out_type=jax.ShapeDtypeStruct((BATCH * OUT_F,), jnp.float32),
    scratch_types=[
        pltpu.VMEM((LOOKUPS,), jnp.float32),
        pltpu.VMEM((B_PER_W, IN_F), jnp.int32),
        pltpu.VMEM((LOOKUPS,), jnp.int32),
        pltpu.VMEM((LOOKUPS,), jnp.float32),
        pltpu.VMEM((LOOKUPS,), jnp.float32),
        pltpu.VMEM((LOOKUPS,), jnp.float32),
        pltpu.VMEM((LOOKUPS,), jnp.float32),
        pltpu.VMEM((IN_F, 128), jnp.int32),
        pltpu.VMEM((IN_F, 128), jnp.int32),
        pltpu.VMEM((IN_F, 128), jnp.int32),
        pltpu.VMEM((IN_F, 128), jnp.int32),
        pltpu.VMEM((LOOKUPS,), jnp.float32),
        pltpu.SemaphoreType.DMA,
        pltpu.SemaphoreType.DMA,
        pltpu.SemaphoreType.DMA,
        pltpu.SemaphoreType.DMA,
    ],
)
def _sc_piecewise(x_hbm, t_hbm, out_hbm, x_v, idx_v, off_v, b0_v, b1_v, b2_v,
                  b3_v, buf0, buf1, buf2, buf3, out_v, sem0, sem1, sem2, sem3):
    bufs = (buf0, buf1, buf2, buf3)
    sems = (sem0, sem1, sem2, sem3)
    bas = (b0_v, b1_v, b2_v, b3_v)

    cid = lax.axis_index("c")
    sid = lax.axis_index("s")
    wid = sid * 2 + cid
    base = wid * LOOKUPS

    pltpu.sync_copy(x_hbm.at[pl.ds(base, LOOKUPS)], x_v)

    iota = lax.iota(jnp.int32, LANES)

    def seg_id(xg):
        u = xg * 256.0 + 256.0
        return jnp.minimum(jnp.maximum(u.astype(jnp.int32), 0), SEGS - 1)

    def idx_body(g, _):
        xg = x_v[pl.ds(g * LANES, LANES)]
        idv = seg_id(xg)
        ilane = (g % 2) * LANES + iota
        r = ilane * N_COEF + 3 * idv
        b = g // 2
        col0 = (g % 2) * LANES
        idx_v[b, pl.ds(col0, LANES)] = r >> 2
        off_v[pl.ds(g * LANES, LANES)] = (r & 3) * LANES
        return 0

    lax.fori_loop(0, LOOKUPS // LANES, idx_body, 0)

    def fire(b, k):
        pltpu.async_copy(t_hbm.at[idx_v.at[b]], bufs[k], sems[k])

    for p in range(NBUF):
        fire(p, p)

    def bas_body(g, _):
        xg = x_v[pl.ds(g * LANES, LANES)]
        idv = seg_id(xg)
        xm = idv.astype(jnp.float32) * (1.0 / 256.0) - 1.0
        t = (xg - xm) * 512.0 - 1.0
        d0 = t + 1.0
        d1 = t + 0.5
        d2 = t - 0.5
        d3 = t - 1.0
        b0_v[pl.ds(g * LANES, LANES)] = d1 * d2 * d3 * (-2.0 / 3.0)
        b1_v[pl.ds(g * LANES, LANES)] = d0 * d2 * d3 * (4.0 / 3.0)
        b2_v[pl.ds(g * LANES, LANES)] = d0 * d1 * d3 * (-4.0 / 3.0)
        b3_v[pl.ds(g * LANES, LANES)] = d0 * d1 * d2 * (2.0 / 3.0)
        return 0

    lax.fori_loop(0, LOOKUPS // LANES, bas_body, 0)

    def b_step(b, k):
        pltpu.make_async_copy(t_hbm.at[idx_v.at[0]], bufs[k], sems[k]).wait()
        rbuf = bufs[k]
        nbase = b * IN_F
        bv = [[bas[j][pl.ds(nbase + h * LANES, LANES)] for h in range(2)]
              for j in range(N_BASIS)]
        qv = [off_v[pl.ds(nbase + h * LANES, LANES)] for h in range(2)]
        accs = [jnp.zeros((LANES,), jnp.float32) for _ in range(8)]
        for i in range(IN_F):
            q0 = qv[i // LANES][i % LANES]
            for j in range(N_BASIS):
                s = bv[j][i // LANES][i % LANES]
                col = pl.multiple_of(q0 + j * LANES, LANES)
                vi = rbuf[i, pl.ds(col, LANES)]
                vlo = lax.bitcast_convert_type(vi << 16, jnp.float32)
                vhi = lax.bitcast_convert_type(vi & jnp.int32(-65536),
                                               jnp.float32)
                accs[2 * j] += s * vlo
                accs[2 * j + 1] += s * vhi
        lo = (accs[0] + accs[2]) + (accs[4] + accs[6])
        hi = (accs[1] + accs[3]) + (accs[5] + accs[7])
        out_v[pl.ds(b * OUT_F, LANES)] = lo
        out_v[pl.ds(b * OUT_F + LANES, LANES)] = hi

        @pl.when(b + NBUF < B_PER_W)
        def _():
            fire(b + NBUF, k)

    def outer(q, _):
        for k in range(NBUF):
            b_step(q * NBUF + k, k)
        return 0

    lax.fori_loop(0, B_PER_W // NBUF, outer, 0)

    pltpu.sync_copy(out_v, out_hbm.at[pl.ds(base, LOOKUPS)])


def kernel(x, w):
    perm = jnp.arange(OUT_F).reshape(2, OUT_F // 2).T.reshape(-1)
    wb = w.astype(jnp.bfloat16)[perm]
    tb = jnp.transpose(wb, (1, 2, 0)).reshape(TBL_ROWS, 128)
    tnext = jnp.concatenate(
        [tb[1:], jnp.zeros((1, 128), jnp.bfloat16)], axis=0)
    tdup = jnp.concatenate([tb, tnext], axis=1)
    ti = jax.lax.bitcast_convert_type(
        tdup.reshape(TBL_ROWS, 128, 2), jnp.int32)
    out_flat = _sc_piecewise(x.reshape(-1), ti)
    return out_flat.reshape(BATCH, OUT_F)


# speedup vs baseline: 5.7976x; 5.7976x over previous
import functools

import jax
import jax.numpy as jnp
from jax import lax
from jax.experimental import pallas as pl
from jax.experimental.pallas import tpu as pltpu
from jax.experimental.pallas import tpu_sc as plsc

N_BASIS = 4
SEGS = 512
IN_F = 32
OUT_F = 32
BATCH = 4096
N_COEF = (N_BASIS - 1) * SEGS + 1
TBL_ROWS = IN_F * N_COEF // 4

NW = 32
B_PER_W = BATCH // NW
LOOKUPS = B_PER_W * IN_F
NBUF = 4
LANES = 16

_mesh = plsc.VectorSubcoreMesh(core_axis_name="c", subcore_axis_name="s")


@functools.partial(
    pl.kernel,
    mesh=_mesh,
    out_type=jax.ShapeDtypeStruct((BATCH * OUT_F,), jnp.float32),
    scratch_types=[
        pltpu.VMEM((LOOKUPS,), jnp.float32),
        pltpu.VMEM((B_PER_W, IN_F), jnp.int32),
        pltpu.VMEM((LOOKUPS,), jnp.int32),
        pltpu.VMEM((LOOKUPS,), jnp.float32),
        pltpu.VMEM((LOOKUPS,), jnp.float32),
        pltpu.VMEM((LOOKUPS,), jnp.float32),
        pltpu.VMEM((LOOKUPS,), jnp.float32),
        pltpu.VMEM((IN_F, 128), jnp.int32),
        pltpu.VMEM((IN_F, 128), jnp.int32),
        pltpu.VMEM((IN_F, 128), jnp.int32),
        pltpu.VMEM((IN_F, 128), jnp.int32),
        pltpu.VMEM((LOOKUPS,), jnp.float32),
        pltpu.SemaphoreType.DMA,
        pltpu.SemaphoreType.DMA,
        pltpu.SemaphoreType.DMA,
        pltpu.SemaphoreType.DMA,
    ],
)
def _sc_piecewise(x_hbm, t_hbm, out_hbm, x_v, idx_v, off_v, b0_v, b1_v, b2_v,
                  b3_v, buf0, buf1, buf2, buf3, out_v, sem0, sem1, sem2, sem3):
    bufs = (buf0, buf1, buf2, buf3)
    sems = (sem0, sem1, sem2, sem3)
    bas = (b0_v, b1_v, b2_v, b3_v)

    cid = lax.axis_index("c")
    sid = lax.axis_index("s")
    wid = sid * 2 + cid
    base = wid * LOOKUPS

    pltpu.sync_copy(x_hbm.at[pl.ds(base, LOOKUPS)], x_v)

    iota = lax.iota(jnp.int32, LANES)

    def seg_id(xg):
        u = xg * 256.0 + 256.0
        return jnp.minimum(jnp.maximum(u.astype(jnp.int32), 0), SEGS - 1)

    def idx_body(g, _):
        xg = x_v[pl.ds(g * LANES, LANES)]
        idv = seg_id(xg)
        ilane = (g % 2) * LANES + iota
        r = ilane * N_COEF + 3 * idv
        b = g // 2
        col0 = (g % 2) * LANES
        idx_v[b, pl.ds(col0, LANES)] = r >> 2
        off_v[pl.ds(g * LANES, LANES)] = (r & 3) * LANES
        return 0

    lax.fori_loop(0, LOOKUPS // LANES, idx_body, 0)

    def fire(b, k):
        pltpu.async_copy(t_hbm.at[idx_v.at[b]], bufs[k], sems[k])

    for p in range(NBUF):
        fire(p, p)

    def bas_body(g, _):
        xg = x_v[pl.ds(g * LANES, LANES)]
        idv = seg_id(xg)
        xm = idv.astype(jnp.float32) * (1.0 / 256.0) - 1.0
        t = (xg - xm) * 512.0 - 1.0
        d0 = t + 1.0
        d1 = t + 0.5
        d2 = t - 0.5
        d3 = t - 1.0
        b0_v[pl.ds(g * LANES, LANES)] = d1 * d2 * d3 * (-2.0 / 3.0)
        b1_v[pl.ds(g * LANES, LANES)] = d0 * d2 * d3 * (4.0 / 3.0)
        b2_v[pl.ds(g * LANES, LANES)] = d0 * d1 * d3 * (-4.0 / 3.0)
        b3_v[pl.ds(g * LANES, LANES)] = d0 * d1 * d2 * (2.0 / 3.0)
        return 0

    lax.fori_loop(0, LOOKUPS // LANES, bas_body, 0)

    def b_step(b, k):
        pltpu.make_async_copy(t_hbm.at[idx_v.at[0]], bufs[k], sems[k]).wait()
        rbuf = bufs[k]
        nbase = b * IN_F
        bv = [[bas[j][pl.ds(nbase + h * LANES, LANES)] for h in range(2)]
              for j in range(N_BASIS)]
        qv = [off_v[pl.ds(nbase + h * LANES, LANES)] for h in range(2)]
        accs = [jnp.zeros((LANES,), jnp.float32) for _ in range(8)]
        for i in range(IN_F):
            q0 = qv[i // LANES][i % LANES]
            for j in range(N_BASIS):
                s = bv[j][i // LANES][i % LANES]
                col = pl.multiple_of(q0 + j * LANES, LANES)
                vi = rbuf[i, pl.ds(col, LANES)]
                vlo = lax.bitcast_convert_type(vi << 16, jnp.float32)
                vhi = lax.bitcast_convert_type(vi & jnp.int32(-65536),
                                               jnp.float32)
                accs[2 * j] += s * vlo
                accs[2 * j + 1] += s * vhi
        lo = (accs[0] + accs[2]) + (accs[4] + accs[6])
        hi = (accs[1] + accs[3]) + (accs[5] + accs[7])
        out_v[pl.ds(b * OUT_F, LANES)] = lo
        out_v[pl.ds(b * OUT_F + LANES, LANES)] = hi

        @pl.when(b + NBUF < B_PER_W)
        def _():
            fire(b + NBUF, k)

    def outer(q, _):
        for k in range(NBUF):
            b_step(q * NBUF + k, k)
        return 0

    lax.fori_loop(0, B_PER_W // NBUF, outer, 0)

    pltpu.sync_copy(out_v, out_hbm.at[pl.ds(base, LOOKUPS)])


def kernel(x, w):
    lo = lax.bitcast_convert_type(
        w[:OUT_F // 2].astype(jnp.bfloat16), jnp.uint16).astype(jnp.int32)
    hi = lax.bitcast_convert_type(
        w[OUT_F // 2:].astype(jnp.bfloat16), jnp.uint16).astype(jnp.int32)
    p = lo | (hi << 16)
    g = jnp.transpose(p, (1, 2, 0)).reshape(TBL_ROWS, 64)
    gnext = jnp.concatenate([g[1:], jnp.zeros((1, 64), jnp.int32)], axis=0)
    ti = jnp.concatenate([g, gnext], axis=1)
    out_flat = _sc_piecewise(x.reshape(-1), ti)
    return out_flat.reshape(BATCH, OUT_F)
